# Initial kernel scaffold; baseline (speedup 1.0000x reference)
import jax
import jax.numpy as jnp
from jax.experimental import pallas as pl


def kernel(pillar_features, coords):
    raise NotImplementedError("write your pallas kernel here")


# speedup vs baseline: 3.0525x; 3.0525x over previous
import functools

import jax
import jax.numpy as jnp
from jax import lax
from jax.experimental import pallas as pl
from jax.experimental.pallas import tpu as pltpu
from jax.experimental.pallas import tpu_sc as plsc

H, W = 496, 432
C = 64
B, P = 4, 12000

NC, NS, L = 2, 16, 16
NW = NC * NS
RR = 16
RSPAN = H // RR
NTASK = B * RR
CG = 4
NG = C // CG
CHUNK = 2000
CH = 128
D16 = 16
MAPN = RSPAN * W
LISTN = P + L


def _iota():
  return lax.iota(jnp.int32, L)


def _popcount(mask):
  return jnp.max(plsc.all_reduce_population_count(mask))


def _sc_body(y_hbm, x_hbm, featv_hbm, out_hbm,
             ybuf, xbuf, cells, pg16, cmap, imgbuf, idxbuf, featbuf, sem):
  wid = lax.axis_index("s") * NC + lax.axis_index("c")
  iota = _iota()
  zeros16 = jnp.zeros((L,), jnp.float32)

  def run_task(t, _):
    task = t * NW + wid
    b = task // RR
    rr = task % RR
    r0 = rr * RSPAN

    def chunk_body(ci, cnt):
      base = ci * CHUNK
      pltpu.sync_copy(y_hbm.at[b, pl.ds(base, CHUNK)], ybuf)
      pltpu.sync_copy(x_hbm.at[b, pl.ds(base, CHUNK)], xbuf)

      def grp(gi, cnt):
        yv = ybuf[pl.ds(gi * L, L)]
        xv = xbuf[pl.ds(gi * L, L)]
        m = ((yv >= r0) & (yv < r0 + RSPAN) & (yv >= 0) & (yv < H)
             & (xv >= 0) & (xv < W))
        cell = (yv - r0) * W + xv
        pg = (b * P + base + gi * L + iota) * (C // D16)
        csum = plsc.cumsum(m.astype(jnp.int32))
        idx = cnt + csum - 1
        plsc.store_scatter(cells, [idx], cell, mask=m)
        plsc.store_scatter(pg16, [idx], pg, mask=m)
        return cnt + jnp.max(csum)

      return lax.fori_loop(0, CHUNK // L, grp, cnt)

    cnt = lax.fori_loop(0, P // CHUNK, chunk_body, jnp.int32(0))

    def mi(i, _):
      cmap[pl.ds(i * L, L)] = jnp.full((L,), -1, jnp.int32)
      return 0

    lax.fori_loop(0, MAPN // L, mi, 0)

    ngrp = (cnt + L - 1) // L

    def dgrp(gi, _):
      sl = gi * L
      cvec = cells[pl.ds(sl, L)]
      slot = sl + iota
      act = (slot < cnt).astype(jnp.int32)

      def cond(pend):
        return jnp.max(pend) > 0

      def body(pend):
        pm = pend > 0
        plsc.store_scatter(cmap, [cvec], slot, mask=pm)
        w = plsc.load_gather(cmap, [cvec], mask=pm)
        return (pm & (w < slot)).astype(jnp.int32)

      lax.while_loop(cond, body, act)
      return 0

    lax.fori_loop(0, ngrp, dgrp, 0)

    def kgrp(gi, cnt2):
      sl = gi * L
      cvec = cells[pl.ds(sl, L)]
      pvec = pg16[pl.ds(sl, L)]
      slot = sl + iota
      act = slot < cnt
      w = plsc.load_gather(cmap, [cvec], mask=act)
      keep = act & (w == slot)
      csum = plsc.cumsum(keep.astype(jnp.int32))
      idx = cnt2 + csum - 1
      plsc.store_scatter(cells, [idx], cvec, mask=keep)
      plsc.store_scatter(pg16, [idx], pvec, mask=keep)
      return cnt2 + jnp.max(csum)

    cnt2 = lax.fori_loop(0, ngrp, kgrp, jnp.int32(0))

    def zi(i, _):
      imgbuf[pl.ds(i * L, L)] = zeros16
      return 0

    lax.fori_loop(0, CG * MAPN // L, zi, 0)

    nch = (cnt2 + CH - 1) // CH

    def group_body(g, _):
      def chunk_b(ch, _):
        m0 = ch * CH

        def ib(q, _):
          sl = m0 + q * L
          pv = pg16[pl.ds(sl, L)]
          act = (sl + iota) < cnt2
          idxbuf[pl.ds(q * L, L)] = jnp.where(act, pv + g // CG, 0)
          return 0

        lax.fori_loop(0, CH // L, ib, 0)
        pltpu.async_copy(featv_hbm.at[idxbuf], featbuf, sem).wait()

        nq = (jnp.minimum(CH, cnt2 - m0) + CG - 1) // CG

        def sq(q, _):
          mi16 = q * CG + iota // CG
          mb = m0 + mi16
          cvec = plsc.load_gather(cells, [mb])
          act = mb < cnt2
          addr = cvec + (iota % CG) * MAPN
          vals = plsc.load_gather(featbuf, [mi16, (g % CG) * CG + iota % CG])
          plsc.store_scatter(imgbuf, [addr], vals, mask=act)
          return 0

        lax.fori_loop(0, nq, sq, 0)
        return 0

      lax.fori_loop(0, nch, chunk_b, 0)

      obase = ((b * C + g * CG) * H + r0) * W
      for cc in range(CG):
        pltpu.sync_copy(imgbuf.at[pl.ds(cc * MAPN, MAPN)],
                        out_hbm.at[pl.ds(obase + cc * H * W, MAPN)])

      def zchunk(ch, _):
        m0 = ch * CH
        nq = (jnp.minimum(CH, cnt2 - m0) + CG - 1) // CG

        def zq(q, _):
          mb = m0 + q * CG + iota // CG
          cvec = plsc.load_gather(cells, [mb])
          act = mb < cnt2
          addr = cvec + (iota % CG) * MAPN
          plsc.store_scatter(imgbuf, [addr], zeros16, mask=act)
          return 0

        lax.fori_loop(0, nq, zq, 0)
        return 0

      lax.fori_loop(0, nch, zchunk, 0)
      return 0

    lax.fori_loop(0, NG, group_body, 0)
    return 0

  lax.fori_loop(0, NTASK // NW, run_task, 0)


@jax.jit
def kernel(pillar_features, coords):
  y32 = coords[:, :, 1].astype(jnp.int32)
  x32 = coords[:, :, 2].astype(jnp.int32)
  featv = pillar_features.reshape(B * P * (C // D16), D16)

  mesh = plsc.VectorSubcoreMesh(core_axis_name="c", subcore_axis_name="s",
                                num_cores=NC, num_subcores=NS)
  f = pl.kernel(
      _sc_body,
      out_type=jax.ShapeDtypeStruct((B * C * H * W,), jnp.float32),
      mesh=mesh,
      compiler_params=pltpu.CompilerParams(use_tc_tiling_on_sc=False,
                                           needs_layout_passes=False),
      scratch_types=[
          pltpu.VMEM((CHUNK,), jnp.int32),
          pltpu.VMEM((CHUNK,), jnp.int32),
          pltpu.VMEM((LISTN,), jnp.int32),
          pltpu.VMEM((LISTN,), jnp.int32),
          pltpu.VMEM((MAPN,), jnp.int32),
          pltpu.VMEM((CG * MAPN,), jnp.float32),
          pltpu.VMEM((CH,), jnp.int32),
          pltpu.VMEM((CH, D16), jnp.float32),
          pltpu.SemaphoreType.DMA,
      ],
  )
  out_flat = f(y32, x32, featv)
  return out_flat.reshape(B, C, H, W)


# speedup vs baseline: 6.5964x; 2.1609x over previous
import functools

import jax
import jax.numpy as jnp
from jax import lax
from jax.experimental import pallas as pl
from jax.experimental.pallas import tpu as pltpu
from jax.experimental.pallas import tpu_sc as plsc

H, W = 496, 432
C = 64
B, P = 4, 12000

NC, NS, L = 2, 16, 16
NW = NC * NS
RR = 16
RSPAN = W // RR
NTASK = B * RR
CG = 4
NG = C // CG
CHUNK = 2000
CH = 128
D16 = 16
MAPN = RSPAN * H
LISTN = P + L


def _iota():
  return lax.iota(jnp.int32, L)


def _popcount(mask):
  return jnp.max(plsc.all_reduce_population_count(mask))


def _sc_body(y_hbm, x_hbm, featv_hbm, out_hbm,
             ybuf, xbuf, cells, pg16, cmap, imgbuf, idxbuf, featbuf, sem):
  wid = lax.axis_index("s") * NC + lax.axis_index("c")
  iota = _iota()
  zeros16 = jnp.zeros((L,), jnp.float32)

  def run_task(t, _):
    task = t * NW + wid
    b = task // RR
    rr = task % RR
    r0 = rr * RSPAN

    def chunk_body(ci, cnt):
      base = ci * CHUNK
      pltpu.sync_copy(y_hbm.at[b, pl.ds(base, CHUNK)], ybuf)
      pltpu.sync_copy(x_hbm.at[b, pl.ds(base, CHUNK)], xbuf)

      def grp(gi, cnt):
        yv = ybuf[pl.ds(gi * L, L)]
        xv = xbuf[pl.ds(gi * L, L)]
        m = ((xv >= r0) & (xv < r0 + RSPAN) & (xv >= 0) & (xv < W)
             & (yv >= 0) & (yv < H))
        cell = (xv - r0) * H + yv
        pg = (b * P + base + gi * L + iota) * (C // D16)
        csum = plsc.cumsum(m.astype(jnp.int32))
        idx = cnt + csum - 1
        plsc.store_scatter(cells, [idx], cell, mask=m)
        plsc.store_scatter(pg16, [idx], pg, mask=m)
        return cnt + jnp.max(csum)

      return lax.fori_loop(0, CHUNK // L, grp, cnt)

    cnt = lax.fori_loop(0, P // CHUNK, chunk_body, jnp.int32(0))

    def mi(i, _):
      cmap[pl.ds(i * L, L)] = jnp.full((L,), -1, jnp.int32)
      return 0

    lax.fori_loop(0, MAPN // L, mi, 0)

    ngrp = (cnt + L - 1) // L

    def dgrp(gi, _):
      sl = gi * L
      cvec = cells[pl.ds(sl, L)]
      slot = sl + iota
      act = (slot < cnt).astype(jnp.int32)

      def cond(pend):
        return jnp.max(pend) > 0

      def body(pend):
        pm = pend > 0
        plsc.store_scatter(cmap, [cvec], slot, mask=pm)
        w = plsc.load_gather(cmap, [cvec], mask=pm)
        return (pm & (w < slot)).astype(jnp.int32)

      lax.while_loop(cond, body, act)
      return 0

    lax.fori_loop(0, ngrp, dgrp, 0)

    def kgrp(gi, cnt2):
      sl = gi * L
      cvec = cells[pl.ds(sl, L)]
      pvec = pg16[pl.ds(sl, L)]
      slot = sl + iota
      act = slot < cnt
      w = plsc.load_gather(cmap, [cvec], mask=act)
      keep = act & (w == slot)
      csum = plsc.cumsum(keep.astype(jnp.int32))
      idx = cnt2 + csum - 1
      plsc.store_scatter(cells, [idx], cvec, mask=keep)
      plsc.store_scatter(pg16, [idx], pvec, mask=keep)
      return cnt2 + jnp.max(csum)

    cnt2 = lax.fori_loop(0, ngrp, kgrp, jnp.int32(0))

    def zi(i, _):
      imgbuf[pl.ds(i * L, L)] = zeros16
      return 0

    lax.fori_loop(0, CG * MAPN // L, zi, 0)

    nch = (cnt2 + CH - 1) // CH

    def group_body(g, _):
      def chunk_b(ch, _):
        m0 = ch * CH

        def ib(q, _):
          sl = m0 + q * L
          pv = pg16[pl.ds(sl, L)]
          act = (sl + iota) < cnt2
          idxbuf[pl.ds(q * L, L)] = jnp.where(act, pv + g // CG, 0)
          return 0

        lax.fori_loop(0, CH // L, ib, 0)
        pltpu.async_copy(featv_hbm.at[idxbuf], featbuf, sem).wait()

        nq = (jnp.minimum(CH, cnt2 - m0) + CG - 1) // CG

        def sq(q, _):
          mi16 = q * CG + iota // CG
          mb = m0 + mi16
          cvec = plsc.load_gather(cells, [mb])
          act = mb < cnt2
          addr = cvec + (iota % CG) * MAPN
          vals = plsc.load_gather(featbuf, [mi16, (g % CG) * CG + iota % CG])
          plsc.store_scatter(imgbuf, [addr], vals, mask=act)
          return 0

        lax.fori_loop(0, nq, sq, 0)
        return 0

      lax.fori_loop(0, nch, chunk_b, 0)

      obase = ((b * C + g * CG) * W + r0) * H
      for cc in range(CG):
        pltpu.sync_copy(imgbuf.at[pl.ds(cc * MAPN, MAPN)],
                        out_hbm.at[pl.ds(obase + cc * H * W, MAPN)])

      def zchunk(ch, _):
        m0 = ch * CH
        nq = (jnp.minimum(CH, cnt2 - m0) + CG - 1) // CG

        def zq(q, _):
          mb = m0 + q * CG + iota // CG
          cvec = plsc.load_gather(cells, [mb])
          act = mb < cnt2
          addr = cvec + (iota % CG) * MAPN
          plsc.store_scatter(imgbuf, [addr], zeros16, mask=act)
          return 0

        lax.fori_loop(0, nq, zq, 0)
        return 0

      lax.fori_loop(0, nch, zchunk, 0)
      return 0

    lax.fori_loop(0, NG, group_body, 0)
    return 0

  lax.fori_loop(0, NTASK // NW, run_task, 0)


@jax.jit
def kernel(pillar_features, coords):
  y32 = coords[:, :, 1].astype(jnp.int32)
  x32 = coords[:, :, 2].astype(jnp.int32)
  featv = pillar_features.reshape(B * P * (C // D16), D16)

  mesh = plsc.VectorSubcoreMesh(core_axis_name="c", subcore_axis_name="s",
                                num_cores=NC, num_subcores=NS)
  f = pl.kernel(
      _sc_body,
      out_type=jax.ShapeDtypeStruct((B * C * H * W,), jnp.float32),
      mesh=mesh,
      compiler_params=pltpu.CompilerParams(use_tc_tiling_on_sc=False,
                                           needs_layout_passes=False),
      scratch_types=[
          pltpu.VMEM((CHUNK,), jnp.int32),
          pltpu.VMEM((CHUNK,), jnp.int32),
          pltpu.VMEM((LISTN,), jnp.int32),
          pltpu.VMEM((LISTN,), jnp.int32),
          pltpu.VMEM((MAPN,), jnp.int32),
          pltpu.VMEM((CG * MAPN,), jnp.float32),
          pltpu.VMEM((CH,), jnp.int32),
          pltpu.VMEM((CH, D16), jnp.float32),
          pltpu.SemaphoreType.DMA,
      ],
  )
  out_flat = f(y32, x32, featv)
  return out_flat.reshape(B, C, W, H).transpose(0, 1, 3, 2)


# speedup vs baseline: 8.5497x; 1.2961x over previous
import jax
import jax.numpy as jnp
from jax import lax
from jax.experimental import pallas as pl
from jax.experimental.pallas import tpu as pltpu
from jax.experimental.pallas import tpu_sc as plsc

H, W = 496, 432
C = 64
B, P = 4, 12000

NC, NS, L = 2, 16, 16
NW = NC * NS
RR = 48
XSPAN = W // RR
NTASK = B * RR
SG = 4
CHUNK = 4000
CH = 128
NSLOT = 8
D16 = 16
MAPN = XSPAN * H
LISTN = P + L


def _sc_body(y_hbm, x_hbm, featv_hbm, out_hbm,
             ybuf, xbuf, cells, pg16, cmap, imgbuf, idxbuf, featbuf,
             g0, g1, g2, g3, g4, g5, g6, g7, sem_s):
  gsems = (g0, g1, g2, g3, g4, g5, g6, g7)
  wid = lax.axis_index("s") * NC + lax.axis_index("c")
  iota = lax.iota(jnp.int32, L)
  iota_map = iota * MAPN
  zeros16 = jnp.zeros((L,), jnp.float32)

  def splat(v):
    return jnp.full((L,), v, jnp.int32)

  def build_idx(slot, chk, sg, cnt2):
    m0 = chk * CH

    def ib(qi, _):
      sl = m0 + qi * L
      pv = pg16[pl.ds(sl, L)]
      act = (sl + iota) < cnt2
      idxbuf[pl.ds(slot * CH + qi * L, L)] = jnp.where(act, pv + sg, 0)
      return 0

    lax.fori_loop(0, CH // L, ib, 0)

  def fire(slot, sem, cnt2=None, chk=None, sg=None):
    if cnt2 is not None:
      build_idx(slot, chk, sg, cnt2)
    return pltpu.async_copy(
        featv_hbm.at[idxbuf.at[pl.ds(slot * CH, CH)]],
        featbuf.at[slot], sem)

  def drain_gather(slot, sem):
    pltpu.make_async_copy(
        featv_hbm.at[idxbuf.at[pl.ds(slot * CH, CH)]],
        featbuf.at[slot], sem).wait()

  def drain_streams():
    pltpu.make_async_copy(
        imgbuf, out_hbm.at[pl.ds(0, D16 * MAPN)], sem_s).wait()

  def run_task(t, _):
    task = t * NW + wid
    b = task // RR
    rr = task % RR
    r0 = rr * XSPAN

    def chunk_body(ci, cntv):
      base = ci * CHUNK
      pltpu.sync_copy(y_hbm.at[b, pl.ds(base, CHUNK)], ybuf)
      pltpu.sync_copy(x_hbm.at[b, pl.ds(base, CHUNK)], xbuf)

      def grp(gi, cntv):
        yv = ybuf[pl.ds(gi * L, L)]
        xv = xbuf[pl.ds(gi * L, L)]
        m = ((xv >= r0) & (xv < r0 + XSPAN) & (xv >= 0) & (xv < W)
             & (yv >= 0) & (yv < H))
        cell = (xv - r0) * H + yv
        pg = (b * P + base + gi * L + iota) * SG
        csum = plsc.cumsum(m.astype(jnp.int32))
        idx = cntv + csum - 1
        plsc.store_scatter(cells, [idx], cell, mask=m)
        plsc.store_scatter(pg16, [idx], pg, mask=m)
        return cntv + plsc.all_reduce_population_count(m)

      return lax.fori_loop(0, CHUNK // L, grp, cntv)

    cntv = lax.fori_loop(0, P // CHUNK, chunk_body, splat(0))
    cnt = jnp.max(cntv)

    def mi(i, _):
      cmap[pl.ds(i * L, L)] = jnp.full((L,), -1, jnp.int32)
      return 0

    lax.fori_loop(0, MAPN // L, mi, 0)

    ngrp = (cnt + L - 1) // L

    def dgrp(gi, _):
      sl = gi * L
      cvec = cells[pl.ds(sl, L)]
      slot = sl + iota
      act = (slot < cnt).astype(jnp.int32)

      def cond(pend):
        return jnp.max(pend) > 0

      def body(pend):
        pm = pend > 0
        plsc.store_scatter(cmap, [cvec], slot, mask=pm)
        w = plsc.load_gather(cmap, [cvec], mask=pm)
        return (pm & (w < slot)).astype(jnp.int32)

      lax.while_loop(cond, body, act)
      return 0

    lax.fori_loop(0, ngrp, dgrp, 0)

    def kgrp(gi, cnt2v):
      sl = gi * L
      cvec = cells[pl.ds(sl, L)]
      pvec = pg16[pl.ds(sl, L)]
      slot = sl + iota
      act = slot < cnt
      w = plsc.load_gather(cmap, [cvec], mask=act)
      keep = act & (w == slot)
      csum = plsc.cumsum(keep.astype(jnp.int32))
      idx = cnt2v + csum - 1
      plsc.store_scatter(cells, [idx], cvec, mask=keep)
      plsc.store_scatter(pg16, [idx], pvec, mask=keep)
      return cnt2v + plsc.all_reduce_population_count(keep)

    cnt2 = jnp.max(lax.fori_loop(0, ngrp, kgrp, splat(0)))
    nch = (cnt2 + CH - 1) // CH

    for k in range(NSLOT):
      sgi, chk = k // 2, k % 2

      @pl.when(chk < nch)
      def _(k=k, sgi=sgi, chk=chk):
        fire(k, gsems[k], cnt2=cnt2, chk=chk, sg=sgi)

    for sgi in range(SG):
      if sgi > 0:
        drain_streams()

      def chunk_loop(ch, _, sgi=sgi):
        par = ch & 1
        refire = ch >= 2

        for parb in range(2):
          slot = sgi * 2 + parb

          @pl.when(refire & (par == parb))
          def _(slot=slot):
            fire(slot, gsems[slot], cnt2=cnt2, chk=ch, sg=sgi).wait()

          @pl.when((~refire) & (par == parb))
          def _(slot=slot):
            drain_gather(slot, gsems[slot])

        slotv = sgi * 2 + par
        mcnt = jnp.minimum(CH, cnt2 - ch * CH)
        slots = splat(slotv)

        def sq(q, _):
          cellv = plsc.load_gather(cells, [splat(ch * CH + q)])
          vals = plsc.load_gather(featbuf, [slots, splat(q), iota])
          plsc.store_scatter(imgbuf, [cellv + iota_map], vals)
          return 0

        lax.fori_loop(0, mcnt, sq, 0)
        return 0

      lax.fori_loop(0, nch, chunk_loop, 0)

      for cc in range(D16):
        obase = ((b * C + sgi * D16 + cc) * W + r0) * H
        pltpu.async_copy(imgbuf.at[pl.ds(cc * MAPN, MAPN)],
                         out_hbm.at[pl.ds(obase, MAPN)], sem_s)

    drain_streams()

    def zq(q, _):
      cellv = plsc.load_gather(cells, [splat(q)])
      plsc.store_scatter(imgbuf, [cellv + iota_map], zeros16)
      return 0

    lax.fori_loop(0, cnt2, zq, 0)
    return 0

  def zi(i, _):
    imgbuf[pl.ds(i * L, L)] = zeros16
    return 0

  lax.fori_loop(0, D16 * MAPN // L, zi, 0)
  lax.fori_loop(0, NTASK // NW, run_task, 0)


@jax.jit
def kernel(pillar_features, coords):
  y32 = coords[:, :, 1].astype(jnp.int32)
  x32 = coords[:, :, 2].astype(jnp.int32)
  featv = pillar_features.reshape(B * P * (C // D16), D16)

  mesh = plsc.VectorSubcoreMesh(core_axis_name="c", subcore_axis_name="s",
                                num_cores=NC, num_subcores=NS)
  f = pl.kernel(
      _sc_body,
      out_type=jax.ShapeDtypeStruct((B * C * H * W,), jnp.float32),
      mesh=mesh,
      compiler_params=pltpu.CompilerParams(use_tc_tiling_on_sc=False,
                                           needs_layout_passes=False),
      scratch_types=[
          pltpu.VMEM((CHUNK,), jnp.int32),
          pltpu.VMEM((CHUNK,), jnp.int32),
          pltpu.VMEM((LISTN,), jnp.int32),
          pltpu.VMEM((LISTN,), jnp.int32),
          pltpu.VMEM((MAPN,), jnp.int32),
          pltpu.VMEM((D16 * MAPN,), jnp.float32),
          pltpu.VMEM((NSLOT * CH,), jnp.int32),
          pltpu.VMEM((NSLOT, CH, D16), jnp.float32),
          pltpu.SemaphoreType.DMA,
          pltpu.SemaphoreType.DMA,
          pltpu.SemaphoreType.DMA,
          pltpu.SemaphoreType.DMA,
          pltpu.SemaphoreType.DMA,
          pltpu.SemaphoreType.DMA,
          pltpu.SemaphoreType.DMA,
          pltpu.SemaphoreType.DMA,
          pltpu.SemaphoreType.DMA,
      ],
  )
  out_flat = f(y32, x32, featv)
  return out_flat.reshape(B, C, W, H).transpose(0, 1, 3, 2)
